# baseline (speedup 1.0000x reference)
import functools

import jax
import jax.numpy as jnp
import numpy as np
from jax import lax
from jax.experimental import pallas as pl
from jax.experimental.pallas import tpu as pltpu
from jax.experimental.pallas import tpu_sc as plsc

NC, NS, L = 2, 16, 16


def _largest_divisor(n, cap):
    for v in range(min(n, cap), 0, -1):
        if n % v == 0:
            return v
    return 1


@functools.lru_cache(maxsize=None)
def _build(n_nodes, n_edges, n_ch, d_node, d_sh, d_edge, hidden, be, bn):
    F = n_ch * d_node * d_sh
    CD = n_ch * d_node
    assert F == 4 * L and CD == 2 * L and d_sh == 2

    P8 = 8
    KE = P8 * d_edge
    KH = P8 * hidden

    def mix_body(ef_ref, ea_ref, w1_ref, w2_ref, w3_ref, w4_ref, sel_ref, a_ref):
        h = ef_ref[...]
        s1 = np.float32(1.0 / np.sqrt(d_edge))
        s2 = np.float32(1.0 / np.sqrt(hidden))
        h = jax.nn.silu(jnp.dot(h, w1_ref[...], preferred_element_type=jnp.float32) * s1)
        h = jax.nn.silu(jnp.dot(h, w2_ref[...], preferred_element_type=jnp.float32) * s2)
        h = jax.nn.silu(jnp.dot(h, w3_ref[...], preferred_element_type=jnp.float32) * s2)
        h = jnp.dot(h, w4_ref[...], preferred_element_type=jnp.float32)
        eab = jnp.dot(ea_ref[...], sel_ref[...], preferred_element_type=jnp.float32)
        a_ref[...] = h * eab

    rows = n_edges // P8
    assert rows % be == 0
    mix = pl.pallas_call(
        mix_body,
        grid=(rows // be,),
        in_specs=[
            pl.BlockSpec((be, KE), lambda i: (i, 0)),
            pl.BlockSpec((be, P8 * d_sh), lambda i: (i, 0)),
            pl.BlockSpec((KE, KH), lambda i: (0, 0)),
            pl.BlockSpec((KH, KH), lambda i: (0, 0)),
            pl.BlockSpec((KH, KH), lambda i: (0, 0)),
            pl.BlockSpec((KH, KH), lambda i: (0, 0)),
            pl.BlockSpec((P8 * d_sh, KH), lambda i: (0, 0)),
        ],
        out_specs=pl.BlockSpec((be, KH), lambda i: (i, 0)),
        out_shape=jax.ShapeDtypeStruct((rows, KH), jnp.float32),
    )

    half = n_nodes // NC
    assert half * NC == n_nodes and half % bn == 0
    unit = np.lcm(NS * 8, bn)
    hpad = int(-(-(half + 1) // unit) * unit)
    rpt = hpad // NS
    ept = n_edges // NS
    assert ept * NS == n_edges
    assert ept % L == 0
    ch = L * _largest_divisor(ept // L, 5)
    nchunk = ept // ch
    assert ch % 8 == 0
    assert nchunk >= 4
    zb = ch
    assert rpt % zb == 0

    mesh = plsc.VectorSubcoreMesh(core_axis_name="c", subcore_axis_name="s")

    scratch = (
        [pltpu.VMEM((ch,), jnp.int32) for _ in range(6)] +
        [pltpu.VMEM((ch, CD), jnp.float32) for _ in range(2)] +
        [pltpu.VMEM((ch, F), jnp.float32) for _ in range(4)] +
        [pltpu.VMEM_SHARED((hpad, F), jnp.float32)] +
        [pltpu.SemaphoreType.DMA for _ in range(8)])

    @functools.partial(
        pl.kernel,
        out_type=jax.ShapeDtypeStruct((NC * hpad, F), jnp.float32),
        mesh=mesh,
        compiler_params=pltpu.CompilerParams(use_tc_tiling_on_sc=False),
        scratch_types=scratch,
    )
    def scatter(nf_hbm, a_hbm, send_hbm, recv_hbm, out_hbm,
                s0, s1, r0, r1, j0, j1, g0, g1, a0, m0, a1, m1, acc,
                is0, is1, gs0, gs1, as0, as1, ss0, ss1):
        sends, recvs, adjs = (s0, s1), (r0, r1), (j0, j1)
        gs, avs, ms = (g0, g1), (a0, a1), (m0, m1)
        i_sems, g_sems = (is0, is1), (gs0, gs1)
        a_sems, s_sems = (as0, as1), (ss0, ss1)
        cid = lax.axis_index("c")
        sid = lax.axis_index("s")
        node_off = cid * half
        zeros = jnp.zeros((L,), jnp.float32)

        def zfill(i, carry):
            m0[i // 4, pl.ds((i % 4) * L, L)] = zeros
            return carry
        lax.fori_loop(0, zb * 4, zfill, 0)
        base_row = sid * rpt

        def zcopy(j, carry):
            pltpu.sync_copy(m0, acc.at[pl.ds(base_row + j * zb, zb)])
            return carry
        lax.fori_loop(0, rpt // zb, zcopy, 0)
        plsc.subcore_barrier()

        ebase = sid * ept

        def issue_idx(k, b):
            off = ebase + k * ch
            pltpu.async_copy(send_hbm.at[pl.ds(off, ch)], sends[b], i_sems[b])
            pltpu.async_copy(recv_hbm.at[pl.ds(off, ch)], recvs[b], i_sems[b])

        def wait_idx(b):
            pltpu.make_async_copy(send_hbm.at[pl.ds(0, ch)], sends[b], i_sems[b]).wait()
            pltpu.make_async_copy(recv_hbm.at[pl.ds(0, ch)], recvs[b], i_sems[b]).wait()

        def issue_fetch(k, b):
            off = ebase + k * ch
            pltpu.async_copy(nf_hbm.at[sends[b]], gs[b], g_sems[b])
            pltpu.async_copy(a_hbm.at[pl.ds(off, ch)], avs[b], a_sems[b])

        def wait_fetch(b):
            pltpu.make_async_copy(nf_hbm.at[sends[b]], gs[b], g_sems[b]).wait()
            pltpu.make_async_copy(a_hbm.at[pl.ds(0, ch)], avs[b], a_sems[b]).wait()

        def issue_scat(b):
            pltpu.async_copy(ms[b], acc.at[adjs[b]], s_sems[b], add=True)

        def wait_scat(b):
            pltpu.make_async_copy(ms[b], acc.at[adjs[b]], s_sems[b]).wait()

        def compute_adj(b):
            def adj(i, c2):
                r = recvs[b][pl.ds(i * L, L)]
                loc = r - node_off
                ok = (loc >= 0) & (loc < half)
                adjs[b][pl.ds(i * L, L)] = jnp.where(ok, loc, half)
                return c2
            lax.fori_loop(0, ch // L, adj, 0)

        def compute_msg(b):
            def msg(e, c2):
                gv0 = gs[b][e, pl.ds(0, L)]
                gv1 = gs[b][e, pl.ds(L, L)]
                ms[b][e, pl.ds(0 * L, L)] = gv0 * avs[b][e, pl.ds(0 * L, L)]
                ms[b][e, pl.ds(1 * L, L)] = gv1 * avs[b][e, pl.ds(1 * L, L)]
                ms[b][e, pl.ds(2 * L, L)] = gv0 * avs[b][e, pl.ds(2 * L, L)]
                ms[b][e, pl.ds(3 * L, L)] = gv1 * avs[b][e, pl.ds(3 * L, L)]
                return c2
            lax.fori_loop(0, ch, msg, 0)

        def body(k, b, first, last):
            wait_fetch(b)
            if not first:
                wait_scat(b)
            compute_adj(b)
            if not last:
                issue_idx(k + 2, b)
            compute_msg(b)
            issue_scat(b)
            if not last:
                wait_idx(b)
                issue_fetch(k + 2, b)

        issue_idx(0, 0)
        issue_idx(1, 1)
        wait_idx(0)
        issue_fetch(0, 0)
        wait_idx(1)
        issue_fetch(1, 1)
        body(0, 0, True, False)
        body(1, 1, True, False)

        p = (nchunk - 4) // 2

        def steady(i, carry):
            k = 2 + 2 * i
            body(k, 0, False, False)
            body(k + 1, 1, False, False)
            return carry
        lax.fori_loop(0, p, steady, 0)

        for k in range(2 + 2 * p, nchunk):
            body(k, k % 2, False, k + 2 >= nchunk)
        wait_scat((nchunk - 2) % 2)
        wait_scat((nchunk - 1) % 2)
        plsc.subcore_barrier()

        def drain(j, carry):
            row0 = base_row + j * zb
            pltpu.sync_copy(acc.at[pl.ds(row0, zb)], a0)
            pltpu.sync_copy(a0, out_hbm.at[pl.ds(cid * hpad + row0, zb)])
            return carry
        lax.fori_loop(0, rpt // zb, drain, 0)

    pad_blocks = (hpad - half) // bn

    def perm_body(x_ref, p_ref, o_ref):
        o_ref[...] = jnp.dot(x_ref[...], p_ref[...],
                             preferred_element_type=jnp.float32)

    def _src_block(i):
        return jnp.where(i < half // bn, i, i + pad_blocks)

    permute = pl.pallas_call(
        perm_body,
        grid=(n_nodes // bn,),
        in_specs=[
            pl.BlockSpec((bn, F), lambda i: (_src_block(i), 0)),
            pl.BlockSpec((F, F), lambda i: (0, 0)),
        ],
        out_specs=pl.BlockSpec((bn, F), lambda i: (i, 0)),
        out_shape=jax.ShapeDtypeStruct((n_nodes, F), jnp.float32),
    )

    return mix, scatter, permute


def kernel(node_feats, edge_attrs, edge_feats, senders, receivers, W1, W2, W3, W4):
    n_nodes, n_ch, d_node = node_feats.shape
    n_edges, d_sh = edge_attrs.shape
    d_edge = edge_feats.shape[1]
    hidden = W2.shape[0]
    F = n_ch * d_node * d_sh
    CD = n_ch * d_node

    p = np.arange(F)
    s, c, d = p // CD, (p % CD) // d_node, p % d_node
    src = c * (d_node * d_sh) + d * d_sh + s
    scale = 1.0 / (np.sqrt(hidden) * np.sqrt(16.0))
    W4P = W4[:, src] * np.float32(scale)

    eye8 = jnp.eye(8, dtype=jnp.float32)
    W1B = jnp.kron(eye8, W1)
    W2B = jnp.kron(eye8, W2)
    W3B = jnp.kron(eye8, W3)
    W4B = jnp.kron(eye8, W4P)
    SEL = np.zeros((8 * d_sh, 8 * F), np.float32)
    for j in range(8):
        SEL[d_sh * j + 0, j * F:j * F + CD] = 1.0
        SEL[d_sh * j + 1, j * F + CD:(j + 1) * F] = 1.0
    SEL = jnp.asarray(SEL)

    j = np.arange(F)
    inv_src = (j % d_sh) * CD + (j // (d_node * d_sh)) * d_node + (j // d_sh) % d_node
    P = np.zeros((F, F), np.float32)
    P[inv_src, j] = 1.0
    P = jnp.asarray(P)

    mix, scatter, permute = _build(n_nodes, n_edges, n_ch, d_node, d_sh,
                                   d_edge, hidden, 1000, 200)
    A = mix(edge_feats.reshape(n_edges // 8, 8 * d_edge),
            edge_attrs.reshape(n_edges // 8, 8 * d_sh),
            W1B, W2B, W3B, W4B, SEL).reshape(n_edges, F)
    acc = scatter(node_feats.reshape(n_nodes, CD), A, senders, receivers)
    out = permute(acc, P)
    return out.reshape(n_nodes, n_ch, d_node * d_sh)


# speedup vs baseline: 1.2695x; 1.2695x over previous
import functools

import jax
import jax.numpy as jnp
import numpy as np
from jax import lax
from jax.experimental import pallas as pl
from jax.experimental.pallas import tpu as pltpu
from jax.experimental.pallas import tpu_sc as plsc

NC, NS, L = 2, 16, 16


def _largest_divisor(n, cap):
    for v in range(min(n, cap), 0, -1):
        if n % v == 0:
            return v
    return 1


@functools.lru_cache(maxsize=None)
def _build(n_nodes, n_edges, n_ch, d_node, d_sh, d_edge, hidden, be, bn):
    F = n_ch * d_node * d_sh
    CD = n_ch * d_node
    assert F == 4 * L and CD == 2 * L and d_sh == 2

    def mix_body(ef_ref, ea_ref, w1_ref, w2_ref, w3_ref, w4_ref, a_ref):
        h = ef_ref[...]
        s1 = np.float32(1.0 / np.sqrt(d_edge))
        s2 = np.float32(1.0 / np.sqrt(hidden))
        h = jax.nn.silu(jnp.dot(h, w1_ref[...], preferred_element_type=jnp.float32) * s1)
        h = jax.nn.silu(jnp.dot(h, w2_ref[...], preferred_element_type=jnp.float32) * s2)
        h = jax.nn.silu(jnp.dot(h, w3_ref[...], preferred_element_type=jnp.float32) * s2)
        h = jnp.dot(h, w4_ref[...], preferred_element_type=jnp.float32)
        ea = ea_ref[...]
        col = lax.broadcasted_iota(jnp.int32, (be, F), 1)
        eab = jnp.where(col < CD, ea[:, 0:1], ea[:, 1:2])
        a_ref[...] = h * eab

    assert n_edges % be == 0
    mix = pl.pallas_call(
        mix_body,
        grid=(n_edges // be,),
        in_specs=[
            pl.BlockSpec((be, d_edge), lambda i: (i, 0)),
            pl.BlockSpec((be, d_sh), lambda i: (i, 0)),
            pl.BlockSpec((d_edge, hidden), lambda i: (0, 0)),
            pl.BlockSpec((hidden, hidden), lambda i: (0, 0)),
            pl.BlockSpec((hidden, hidden), lambda i: (0, 0)),
            pl.BlockSpec((hidden, F), lambda i: (0, 0)),
        ],
        out_specs=pl.BlockSpec((be, F), lambda i: (i, 0)),
        out_shape=jax.ShapeDtypeStruct((n_edges, F), jnp.float32),
    )

    half = n_nodes // NC
    assert half * NC == n_nodes and half % bn == 0
    unit = np.lcm(NS * 8, bn)
    hpad = int(-(-(half + 1) // unit) * unit)
    rpt = hpad // NS
    ept = n_edges // NS
    assert ept * NS == n_edges
    assert ept % L == 0
    ch = L * _largest_divisor(ept // L, 5)
    nchunk = ept // ch
    assert ch % 8 == 0
    assert nchunk >= 4
    zb = ch
    assert rpt % zb == 0

    mesh = plsc.VectorSubcoreMesh(core_axis_name="c", subcore_axis_name="s")

    scratch = (
        [pltpu.VMEM((ch,), jnp.int32) for _ in range(6)] +
        [pltpu.VMEM((ch, CD), jnp.float32) for _ in range(2)] +
        [pltpu.VMEM((ch, F), jnp.float32) for _ in range(4)] +
        [pltpu.VMEM_SHARED((hpad, F), jnp.float32)] +
        [pltpu.SemaphoreType.DMA for _ in range(8)])

    @functools.partial(
        pl.kernel,
        out_type=jax.ShapeDtypeStruct((NC * hpad, F), jnp.float32),
        mesh=mesh,
        compiler_params=pltpu.CompilerParams(use_tc_tiling_on_sc=False),
        scratch_types=scratch,
    )
    def scatter(nf_hbm, a_hbm, send_hbm, recv_hbm, out_hbm,
                s0, s1, r0, r1, j0, j1, g0, g1, a0, m0, a1, m1, acc,
                is0, is1, gs0, gs1, as0, as1, ss0, ss1):
        sends, recvs, adjs = (s0, s1), (r0, r1), (j0, j1)
        gs, avs, ms = (g0, g1), (a0, a1), (m0, m1)
        i_sems, g_sems = (is0, is1), (gs0, gs1)
        a_sems, s_sems = (as0, as1), (ss0, ss1)
        cid = lax.axis_index("c")
        sid = lax.axis_index("s")
        node_off = cid * half
        zeros = jnp.zeros((L,), jnp.float32)

        def zfill(i, carry):
            m0[i // 4, pl.ds((i % 4) * L, L)] = zeros
            return carry
        lax.fori_loop(0, zb * 4, zfill, 0)
        base_row = sid * rpt

        def zcopy(j, carry):
            pltpu.sync_copy(m0, acc.at[pl.ds(base_row + j * zb, zb)])
            return carry
        lax.fori_loop(0, rpt // zb, zcopy, 0)
        plsc.subcore_barrier()

        ebase = sid * ept

        def issue_idx(k, b):
            off = ebase + k * ch
            pltpu.async_copy(send_hbm.at[pl.ds(off, ch)], sends[b], i_sems[b])
            pltpu.async_copy(recv_hbm.at[pl.ds(off, ch)], recvs[b], i_sems[b])

        def wait_idx(b):
            pltpu.make_async_copy(send_hbm.at[pl.ds(0, ch)], sends[b], i_sems[b]).wait()
            pltpu.make_async_copy(recv_hbm.at[pl.ds(0, ch)], recvs[b], i_sems[b]).wait()

        def issue_fetch(k, b):
            off = ebase + k * ch
            pltpu.async_copy(nf_hbm.at[sends[b]], gs[b], g_sems[b])
            pltpu.async_copy(a_hbm.at[pl.ds(off, ch)], avs[b], a_sems[b])

        def wait_fetch(b):
            pltpu.make_async_copy(nf_hbm.at[sends[b]], gs[b], g_sems[b]).wait()
            pltpu.make_async_copy(a_hbm.at[pl.ds(0, ch)], avs[b], a_sems[b]).wait()

        def issue_scat(b):
            pltpu.async_copy(ms[b], acc.at[adjs[b]], s_sems[b], add=True)

        def wait_scat(b):
            pltpu.make_async_copy(ms[b], acc.at[adjs[b]], s_sems[b]).wait()

        def compute_adj(b):
            def adj(i, c2):
                r = recvs[b][pl.ds(i * L, L)]
                loc = r - node_off
                ok = (loc >= 0) & (loc < half)
                adjs[b][pl.ds(i * L, L)] = jnp.where(ok, loc, half)
                return c2
            lax.fori_loop(0, ch // L, adj, 0)

        def compute_msg(b):
            def msg(e, c2):
                gv0 = gs[b][e, pl.ds(0, L)]
                gv1 = gs[b][e, pl.ds(L, L)]
                ms[b][e, pl.ds(0 * L, L)] = gv0 * avs[b][e, pl.ds(0 * L, L)]
                ms[b][e, pl.ds(1 * L, L)] = gv1 * avs[b][e, pl.ds(1 * L, L)]
                ms[b][e, pl.ds(2 * L, L)] = gv0 * avs[b][e, pl.ds(2 * L, L)]
                ms[b][e, pl.ds(3 * L, L)] = gv1 * avs[b][e, pl.ds(3 * L, L)]
                return c2
            lax.fori_loop(0, ch, msg, 0)

        def body(k, b, first, last):
            wait_fetch(b)
            if not first:
                wait_scat(b)
            compute_adj(b)
            if not last:
                issue_idx(k + 2, b)
            compute_msg(b)
            issue_scat(b)
            if not last:
                wait_idx(b)
                issue_fetch(k + 2, b)

        issue_idx(0, 0)
        issue_idx(1, 1)
        wait_idx(0)
        issue_fetch(0, 0)
        wait_idx(1)
        issue_fetch(1, 1)
        body(0, 0, True, False)
        body(1, 1, True, False)

        p = (nchunk - 4) // 2

        def steady(i, carry):
            k = 2 + 2 * i
            body(k, 0, False, False)
            body(k + 1, 1, False, False)
            return carry
        lax.fori_loop(0, p, steady, 0)

        for k in range(2 + 2 * p, nchunk):
            body(k, k % 2, False, k + 2 >= nchunk)
        wait_scat((nchunk - 2) % 2)
        wait_scat((nchunk - 1) % 2)
        plsc.subcore_barrier()

        def drain(j, carry):
            row0 = base_row + j * zb
            pltpu.sync_copy(acc.at[pl.ds(row0, zb)], a0)
            pltpu.sync_copy(a0, out_hbm.at[pl.ds(cid * hpad + row0, zb)])
            return carry
        lax.fori_loop(0, rpt // zb, drain, 0)

    pad_blocks = (hpad - half) // bn

    def perm_body(x_ref, p_ref, o_ref):
        o_ref[...] = jnp.dot(x_ref[...], p_ref[...],
                             preferred_element_type=jnp.float32)

    def _src_block(i):
        return jnp.where(i < half // bn, i, i + pad_blocks)

    permute = pl.pallas_call(
        perm_body,
        grid=(n_nodes // bn,),
        in_specs=[
            pl.BlockSpec((bn, F), lambda i: (_src_block(i), 0)),
            pl.BlockSpec((F, F), lambda i: (0, 0)),
        ],
        out_specs=pl.BlockSpec((bn, F), lambda i: (i, 0)),
        out_shape=jax.ShapeDtypeStruct((n_nodes, F), jnp.float32),
    )

    return mix, scatter, permute


def kernel(node_feats, edge_attrs, edge_feats, senders, receivers, W1, W2, W3, W4):
    n_nodes, n_ch, d_node = node_feats.shape
    n_edges, d_sh = edge_attrs.shape
    d_edge = edge_feats.shape[1]
    hidden = W2.shape[0]
    F = n_ch * d_node * d_sh
    CD = n_ch * d_node

    p = np.arange(F)
    s, c, d = p // CD, (p % CD) // d_node, p % d_node
    src = c * (d_node * d_sh) + d * d_sh + s
    scale = 1.0 / (np.sqrt(hidden) * np.sqrt(16.0))
    W4P = W4[:, src] * np.float32(scale)

    j = np.arange(F)
    inv_src = (j % d_sh) * CD + (j // (d_node * d_sh)) * d_node + (j // d_sh) % d_node
    P = np.zeros((F, F), np.float32)
    P[inv_src, j] = 1.0
    P = jnp.asarray(P)

    mix, scatter, permute = _build(n_nodes, n_edges, n_ch, d_node, d_sh,
                                   d_edge, hidden, 8000, 200)
    A = mix(edge_feats, edge_attrs, W1, W2, W3, W4P)
    acc = scatter(node_feats.reshape(n_nodes, CD), A, senders, receivers)
    out = permute(acc, P)
    return out.reshape(n_nodes, n_ch, d_node * d_sh)


# speedup vs baseline: 1.4405x; 1.1347x over previous
import functools

import jax
import jax.numpy as jnp
import numpy as np
from jax import lax
from jax.experimental import pallas as pl
from jax.experimental.pallas import tpu as pltpu
from jax.experimental.pallas import tpu_sc as plsc

NC, NS, L = 2, 16, 16


@functools.lru_cache(maxsize=None)
def _build(n_nodes, n_edges, n_ch, d_node, d_sh, d_edge, hidden, be, bn):
    F = n_ch * d_node * d_sh
    CD = n_ch * d_node
    assert F == 4 * L and CD == 2 * L and d_sh == 2

    def mix_body(ef_ref, ea_ref, w1_ref, w2_ref, w3_ref, w4_ref, a_ref):
        h = ef_ref[...]
        s1 = np.float32(1.0 / np.sqrt(d_edge))
        s2 = np.float32(1.0 / np.sqrt(hidden))
        h = jax.nn.silu(jnp.dot(h, w1_ref[...], preferred_element_type=jnp.float32) * s1)
        h = jax.nn.silu(jnp.dot(h, w2_ref[...], preferred_element_type=jnp.float32) * s2)
        h = jax.nn.silu(jnp.dot(h, w3_ref[...], preferred_element_type=jnp.float32) * s2)
        h = jnp.dot(h, w4_ref[...], preferred_element_type=jnp.float32)
        ea = ea_ref[...]
        col = lax.broadcasted_iota(jnp.int32, (be, F), 1)
        eab = jnp.where(col < CD, ea[:, 0:1], ea[:, 1:2])
        a_ref[...] = h * eab

    def make_mix(off_blocks, rows):
        assert rows % be == 0
        return pl.pallas_call(
            mix_body,
            grid=(rows // be,),
            in_specs=[
                pl.BlockSpec((be, d_edge), lambda i: (i + off_blocks, 0)),
                pl.BlockSpec((be, d_sh), lambda i: (i + off_blocks, 0)),
                pl.BlockSpec((d_edge, hidden), lambda i: (0, 0)),
                pl.BlockSpec((hidden, hidden), lambda i: (0, 0)),
                pl.BlockSpec((hidden, hidden), lambda i: (0, 0)),
                pl.BlockSpec((hidden, F), lambda i: (0, 0)),
            ],
            out_specs=pl.BlockSpec((be, F), lambda i: (i, 0)),
            out_shape=jax.ShapeDtypeStruct((rows, F), jnp.float32),
        )

    half = n_nodes // NC
    assert half * NC == n_nodes and half % bn == 0
    unit = np.lcm(NS * 8, bn)
    hpad = int(-(-(half + 1) // unit) * unit)
    rpt = hpad // NS
    ch = 80
    zb = ch
    assert rpt % zb == 0

    mesh = plsc.VectorSubcoreMesh(core_axis_name="c", subcore_axis_name="s")

    scratch = (
        [pltpu.VMEM((ch,), jnp.int32) for _ in range(6)] +
        [pltpu.VMEM((ch, CD), jnp.float32) for _ in range(2)] +
        [pltpu.VMEM((ch, F), jnp.float32) for _ in range(4)] +
        [pltpu.VMEM_SHARED((hpad, F), jnp.float32)] +
        [pltpu.SemaphoreType.DMA for _ in range(8)])

    def make_scatter(eoff, nsub):
        ept = nsub // NS
        assert ept * NS == nsub and ept % ch == 0
        nchunk = ept // ch
        assert nchunk >= 4

        @functools.partial(
            pl.kernel,
            out_type=jax.ShapeDtypeStruct((NC * hpad, F), jnp.float32),
            mesh=mesh,
            compiler_params=pltpu.CompilerParams(use_tc_tiling_on_sc=False),
            scratch_types=scratch,
        )
        def scatter(nf_hbm, a_hbm, send_hbm, recv_hbm, out_hbm,
                    s0, s1, r0, r1, j0, j1, g0, g1, a0, m0, a1, m1, acc,
                    is0, is1, gs0, gs1, as0, as1, ss0, ss1):
            sends, recvs, adjs = (s0, s1), (r0, r1), (j0, j1)
            gs, avs, ms = (g0, g1), (a0, a1), (m0, m1)
            i_sems, g_sems = (is0, is1), (gs0, gs1)
            a_sems, s_sems = (as0, as1), (ss0, ss1)
            cid = lax.axis_index("c")
            sid = lax.axis_index("s")
            node_off = cid * half
            zeros = jnp.zeros((L,), jnp.float32)

            def zfill(i, carry):
                m0[i // 4, pl.ds((i % 4) * L, L)] = zeros
                return carry
            lax.fori_loop(0, zb * 4, zfill, 0)
            base_row = sid * rpt

            def zcopy(j, carry):
                pltpu.sync_copy(m0, acc.at[pl.ds(base_row + j * zb, zb)])
                return carry
            lax.fori_loop(0, rpt // zb, zcopy, 0)
            plsc.subcore_barrier()

            ebase = sid * ept

            def issue_idx(k, b):
                off = eoff + ebase + k * ch
                pltpu.async_copy(send_hbm.at[pl.ds(off, ch)], sends[b], i_sems[b])
                pltpu.async_copy(recv_hbm.at[pl.ds(off, ch)], recvs[b], i_sems[b])

            def wait_idx(b):
                pltpu.make_async_copy(send_hbm.at[pl.ds(0, ch)], sends[b], i_sems[b]).wait()
                pltpu.make_async_copy(recv_hbm.at[pl.ds(0, ch)], recvs[b], i_sems[b]).wait()

            def issue_fetch(k, b):
                off = ebase + k * ch
                pltpu.async_copy(nf_hbm.at[sends[b]], gs[b], g_sems[b])
                pltpu.async_copy(a_hbm.at[pl.ds(off, ch)], avs[b], a_sems[b])

            def wait_fetch(b):
                pltpu.make_async_copy(nf_hbm.at[sends[b]], gs[b], g_sems[b]).wait()
                pltpu.make_async_copy(a_hbm.at[pl.ds(0, ch)], avs[b], a_sems[b]).wait()

            def issue_scat(b):
                pltpu.async_copy(ms[b], acc.at[adjs[b]], s_sems[b], add=True)

            def wait_scat(b):
                pltpu.make_async_copy(ms[b], acc.at[adjs[b]], s_sems[b]).wait()

            def compute_adj(b):
                def adj(i, c2):
                    r = recvs[b][pl.ds(i * L, L)]
                    loc = r - node_off
                    ok = (loc >= 0) & (loc < half)
                    adjs[b][pl.ds(i * L, L)] = jnp.where(ok, loc, half)
                    return c2
                lax.fori_loop(0, ch // L, adj, 0)

            def compute_msg(b):
                def msg(e, c2):
                    gv0 = gs[b][e, pl.ds(0, L)]
                    gv1 = gs[b][e, pl.ds(L, L)]
                    ms[b][e, pl.ds(0 * L, L)] = gv0 * avs[b][e, pl.ds(0 * L, L)]
                    ms[b][e, pl.ds(1 * L, L)] = gv1 * avs[b][e, pl.ds(1 * L, L)]
                    ms[b][e, pl.ds(2 * L, L)] = gv0 * avs[b][e, pl.ds(2 * L, L)]
                    ms[b][e, pl.ds(3 * L, L)] = gv1 * avs[b][e, pl.ds(3 * L, L)]
                    return c2
                lax.fori_loop(0, ch, msg, 0)

            def body(k, b, first, last):
                wait_fetch(b)
                if not first:
                    wait_scat(b)
                compute_adj(b)
                if not last:
                    issue_idx(k + 2, b)
                compute_msg(b)
                issue_scat(b)
                if not last:
                    wait_idx(b)
                    issue_fetch(k + 2, b)

            issue_idx(0, 0)
            issue_idx(1, 1)
            wait_idx(0)
            issue_fetch(0, 0)
            wait_idx(1)
            issue_fetch(1, 1)
            body(0, 0, True, False)
            body(1, 1, True, False)

            p = (nchunk - 4) // 2

            def steady(i, carry):
                k = 2 + 2 * i
                body(k, 0, False, False)
                body(k + 1, 1, False, False)
                return carry
            lax.fori_loop(0, p, steady, 0)

            for k in range(2 + 2 * p, nchunk):
                body(k, k % 2, False, k + 2 >= nchunk)
            wait_scat((nchunk - 2) % 2)
            wait_scat((nchunk - 1) % 2)
            plsc.subcore_barrier()

            def drain(j, carry):
                row0 = base_row + j * zb
                pltpu.sync_copy(acc.at[pl.ds(row0, zb)], a0)
                pltpu.sync_copy(a0, out_hbm.at[pl.ds(cid * hpad + row0, zb)])
                return carry
            lax.fori_loop(0, rpt // zb, drain, 0)

        return scatter

    pad_blocks = (hpad - half) // bn

    def perm_body(x_ref, y_ref, p_ref, o_ref):
        o_ref[...] = jnp.dot(x_ref[...] + y_ref[...], p_ref[...],
                             preferred_element_type=jnp.float32)

    def _src_block(i):
        return jnp.where(i < half // bn, i, i + pad_blocks)

    permute = pl.pallas_call(
        perm_body,
        grid=(n_nodes // bn,),
        in_specs=[
            pl.BlockSpec((bn, F), lambda i: (_src_block(i), 0)),
            pl.BlockSpec((bn, F), lambda i: (_src_block(i), 0)),
            pl.BlockSpec((F, F), lambda i: (0, 0)),
        ],
        out_specs=pl.BlockSpec((bn, F), lambda i: (i, 0)),
        out_shape=jax.ShapeDtypeStruct((n_nodes, F), jnp.float32),
    )

    unit2 = int(np.lcm(NS * ch, be))
    e1 = max(unit2, (n_edges // 2 // unit2) * unit2)
    assert 0 < e1 < n_edges and (n_edges - e1) % unit2 == 0
    mix1, mix2 = make_mix(0, e1), make_mix(e1 // be, n_edges - e1)
    sc1, sc2 = make_scatter(0, e1), make_scatter(e1, n_edges - e1)
    return mix1, mix2, sc1, sc2, permute


def kernel(node_feats, edge_attrs, edge_feats, senders, receivers, W1, W2, W3, W4):
    n_nodes, n_ch, d_node = node_feats.shape
    n_edges, d_sh = edge_attrs.shape
    d_edge = edge_feats.shape[1]
    hidden = W2.shape[0]
    F = n_ch * d_node * d_sh
    CD = n_ch * d_node

    p = np.arange(F)
    s, c, d = p // CD, (p % CD) // d_node, p % d_node
    src = c * (d_node * d_sh) + d * d_sh + s
    scale = 1.0 / (np.sqrt(hidden) * np.sqrt(16.0))
    W4P = W4[:, src] * np.float32(scale)

    j = np.arange(F)
    inv_src = (j % d_sh) * CD + (j // (d_node * d_sh)) * d_node + (j // d_sh) % d_node
    P = np.zeros((F, F), np.float32)
    P[inv_src, j] = 1.0
    P = jnp.asarray(P)

    mix1, mix2, sc1, sc2, permute = _build(n_nodes, n_edges, n_ch, d_node,
                                           d_sh, d_edge, hidden, 8000, 200)
    nf2 = node_feats.reshape(n_nodes, CD)
    A1 = mix1(edge_feats, edge_attrs, W1, W2, W3, W4P)
    acc1 = sc1(nf2, A1, senders, receivers)
    A2 = mix2(edge_feats, edge_attrs, W1, W2, W3, W4P)
    acc2 = sc2(nf2, A2, senders, receivers)
    out = permute(acc1, acc2, P)
    return out.reshape(n_nodes, n_ch, d_node * d_sh)
